# baseline (speedup 1.0000x reference)
import functools

import jax
import jax.numpy as jnp
from jax import lax
from jax.experimental import pallas as pl
from jax.experimental.pallas import tpu as pltpu
from jax.experimental.pallas import tpu_sc as plsc

N_NODES = 10000
N_EDGES = 320000
F = 128
FH = F // 2
EPS = 1e-5

NC = 2
NS = 16
EPT = N_EDGES // NS
CH = 80
NCH = EPT // CH
NBUF = 10
GRP = 50
NG = NCH // GRP
N_PAD = 10240
ROWS_PER_TILE = N_PAD // NS


def _sc_spmm(x, edges):
    mesh = plsc.VectorSubcoreMesh(core_axis_name="c", subcore_axis_name="s")
    GE = GRP * CH
    XRT = N_NODES // NS

    @functools.partial(
        pl.kernel,
        mesh=mesh,
        out_type=(
            jax.ShapeDtypeStruct((NC, N_PAD, FH), jnp.float32),
            jax.ShapeDtypeStruct((NC, N_NODES, FH), jnp.float32),
        ),
        scratch_types=[
            pltpu.VMEM_SHARED((N_PAD, FH), jnp.float32),
            pltpu.VMEM((2, GE), jnp.int32),
            pltpu.VMEM((2, GE), jnp.int32),
            pltpu.VMEM((NBUF, CH, FH), jnp.float32),
            pltpu.SemaphoreType.DMA,
            pltpu.SemaphoreType.DMA,
            pltpu.SemaphoreType.DMA,
        ],
        compiler_params=pltpu.CompilerParams(use_tc_tiling_on_sc=False),
    )
    def k(x_hbm, e_hbm, part_hbm, xs_hbm, acc, colb, rowb, rbuf,
          gsem, ssem, isem):
        c = lax.axis_index("c")
        s = lax.axis_index("s")
        ebase = s * EPT

        xr = pl.ds(s * XRT, XRT)

        @pl.when(c == 0)
        def _():
            pltpu.sync_copy(x_hbm.at[xr, pl.ds(0, FH)], xs_hbm.at[0, xr])

        @pl.when(c == 1)
        def _():
            pltpu.sync_copy(x_hbm.at[xr, pl.ds(FH, FH)], xs_hbm.at[1, xr])

        zc = rbuf.at[0]

        def zbody(i, carry):
            for j in range(FH // 16):
                zc[i, pl.ds(j * 16, 16)] = jnp.zeros((16,), jnp.float32)
            return carry

        lax.fori_loop(0, CH, zbody, 0)
        for t in range(ROWS_PER_TILE // CH):
            pltpu.sync_copy(zc, acc.at[pl.ds(s * ROWS_PER_TILE + t * CH, CH)])

        pltpu.sync_copy(e_hbm.at[1, pl.ds(ebase, GE)], colb.at[0])
        pltpu.sync_copy(e_hbm.at[0, pl.ds(ebase, GE)], rowb.at[0])
        plsc.subcore_barrier()

        def make_body(x_hbm, pg):
            def body(o, carry):
                cps = []
                for b in range(NBUF):
                    cps.append(
                        pltpu.async_copy(
                            x_hbm.at[colb.at[pg, pl.ds((o * NBUF + b) * CH, CH)]],
                            rbuf.at[b], gsem,
                        )
                    )
                scs = []
                for b in range(NBUF):
                    cps[b].wait()
                    scs.append(
                        pltpu.async_copy(
                            rbuf.at[b],
                            acc.at[rowb.at[pg, pl.ds((o * NBUF + b) * CH, CH)]],
                            ssem, add=True,
                        )
                    )
                for b in range(NBUF):
                    scs[b].wait()
                return carry

            return body

        for g in range(NG):
            pg = g % 2
            nxt = []
            if g + 1 < NG:
                nb = ebase + (g + 1) * GE
                nxt.append(
                    pltpu.async_copy(e_hbm.at[1, pl.ds(nb, GE)], colb.at[1 - pg], isem)
                )
                nxt.append(
                    pltpu.async_copy(e_hbm.at[0, pl.ds(nb, GE)], rowb.at[1 - pg], isem)
                )

            @pl.when(c == 0)
            def _():
                lax.fori_loop(0, GRP // NBUF, make_body(xs_hbm.at[0], pg), 0)

            @pl.when(c == 1)
            def _():
                lax.fori_loop(0, GRP // NBUF, make_body(xs_hbm.at[1], pg), 0)

            for cp in nxt:
                cp.wait()

        plsc.subcore_barrier()
        sl = pl.ds(s * ROWS_PER_TILE, ROWS_PER_TILE)
        pltpu.sync_copy(acc.at[sl], part_hbm.at[c, sl])

    return k(x, edges)[0]


def _tc_post(partials, weight, gamma, beta):
    def body(p_ref, w_ref, g_ref, b_ref, o_ref):
        e0, e1, e2, e3, e4, e5, e6, e7 = jnp.split(w_ref[...], 8, axis=1)
        e_0 = jnp.concatenate([e0, -e1, -e2, -e3, -e4, -e5, -e6, -e7], axis=0)
        e_1 = jnp.concatenate([e1, e0, -e3, e2, -e5, e4, e7, -e6], axis=0)
        e_2 = jnp.concatenate([e2, e3, e0, -e1, -e6, -e7, e4, e5], axis=0)
        e_3 = jnp.concatenate([e3, -e2, e1, e0, -e7, e6, -e5, e4], axis=0)
        e_4 = jnp.concatenate([e4, e5, e6, e7, e0, -e1, -e2, -e3], axis=0)
        e_5 = jnp.concatenate([e5, -e4, e7, -e6, e1, e0, e3, -e2], axis=0)
        e_6 = jnp.concatenate([e6, -e7, -e4, e5, e2, -e3, e0, e1], axis=0)
        e_7 = jnp.concatenate([e7, e6, -e5, -e4, e3, e2, -e1, e0], axis=0)
        hamilton = jnp.concatenate([e_0, e_1, e_2, e_3, e_4, e_5, e_6, e_7], axis=1)

        agg = jnp.concatenate([p_ref[0], p_ref[1]], axis=1)
        agg = lax.slice(agg, (0, 0), (N_NODES, F))
        out = jnp.dot(agg, hamilton, preferred_element_type=jnp.float32)
        mean = jnp.mean(out, axis=0, keepdims=True)
        var = jnp.mean(jnp.square(out - mean), axis=0, keepdims=True)
        o_ref[...] = jnp.tanh(
            (out - mean) * lax.rsqrt(var + EPS) * g_ref[...] + b_ref[...]
        )

    return pl.pallas_call(
        body,
        out_shape=jax.ShapeDtypeStruct((N_NODES, F), jnp.float32),
    )(partials, weight, gamma.reshape(1, F), beta.reshape(1, F))


def kernel(x, edge_index, weight, gamma, beta):
    partials = _sc_spmm(x, edge_index.astype(jnp.int32))
    return _tc_post(partials, weight, gamma, beta)


# speedup vs baseline: 1.8411x; 1.8411x over previous
import functools

import jax
import jax.numpy as jnp
from jax import lax
from jax.experimental import pallas as pl
from jax.experimental.pallas import tpu as pltpu
from jax.experimental.pallas import tpu_sc as plsc

N_NODES = 10000
N_EDGES = 320000
F = 128
FH = F // 2
EPS = 1e-5

NC = 2
NS = 16
EPT = N_EDGES // NS
CH = 128
NFC = EPT // CH
TAIL = EPT - NFC * CH
NBUF = 6
GROUPS = (24, 24, 24, 24, 24, 24, 12)
GMAX = max(GROUPS)
N_PAD = 10240
ROWS_PER_TILE = N_PAD // NS


def _sc_spmm(x0, x1, edges):
    mesh = plsc.VectorSubcoreMesh(core_axis_name="c", subcore_axis_name="s")

    @functools.partial(
        pl.kernel,
        mesh=mesh,
        out_type=jax.ShapeDtypeStruct((NC, N_PAD, FH), jnp.float32),
        scratch_types=[
            pltpu.VMEM_SHARED((N_PAD, FH), jnp.float32),
            pltpu.VMEM((2, GMAX * CH), jnp.int32),
            pltpu.VMEM((2, GMAX * CH), jnp.int32),
            pltpu.VMEM((NBUF, CH, FH), jnp.float32),
            pltpu.SemaphoreType.DMA,
            pltpu.SemaphoreType.DMA,
            pltpu.SemaphoreType.DMA,
        ],
        compiler_params=pltpu.CompilerParams(use_tc_tiling_on_sc=False),
    )
    def k(x0_hbm, x1_hbm, e_hbm, part_hbm, acc, colb, rowb, rbuf,
          gsem, ssem, isem):
        c = lax.axis_index("c")
        s = lax.axis_index("s")
        ebase = s * EPT

        zc = rbuf.at[0]

        def zbody(i, carry):
            for j in range(FH // 16):
                zc[i, pl.ds(j * 16, 16)] = jnp.zeros((16,), jnp.float32)
            return carry

        lax.fori_loop(0, CH, zbody, 0)
        for t in range(ROWS_PER_TILE // CH):
            pltpu.sync_copy(zc, acc.at[pl.ds(s * ROWS_PER_TILE + t * CH, CH)])

        ge0 = GROUPS[0] * CH
        pltpu.sync_copy(e_hbm.at[1, pl.ds(ebase, ge0)], colb.at[0, pl.ds(0, ge0)])
        pltpu.sync_copy(e_hbm.at[0, pl.ds(ebase, ge0)], rowb.at[0, pl.ds(0, ge0)])
        plsc.subcore_barrier()

        def make_body(x_hbm, pg):
            def body(o, carry):
                cps = []
                for b in range(NBUF):
                    cps.append(
                        pltpu.async_copy(
                            x_hbm.at[colb.at[pg, pl.ds((o * NBUF + b) * CH, CH)]],
                            rbuf.at[b], gsem,
                        )
                    )
                scs = []
                for b in range(NBUF):
                    cps[b].wait()
                    scs.append(
                        pltpu.async_copy(
                            rbuf.at[b],
                            acc.at[rowb.at[pg, pl.ds((o * NBUF + b) * CH, CH)]],
                            ssem, add=True,
                        )
                    )
                for b in range(NBUF):
                    scs[b].wait()
                return carry

            return body

        gbase = 0
        for g, gsz in enumerate(GROUPS):
            pg = g % 2
            nxt = []
            if g + 1 < len(GROUPS):
                nb = ebase + (gbase + gsz) * CH
                ge = GROUPS[g + 1] * CH
                nxt.append(
                    pltpu.async_copy(
                        e_hbm.at[1, pl.ds(nb, ge)], colb.at[1 - pg, pl.ds(0, ge)],
                        isem,
                    )
                )
                nxt.append(
                    pltpu.async_copy(
                        e_hbm.at[0, pl.ds(nb, ge)], rowb.at[1 - pg, pl.ds(0, ge)],
                        isem,
                    )
                )

            @pl.when(c == 0)
            def _():
                lax.fori_loop(0, gsz // NBUF, make_body(x0_hbm, pg), 0)

            @pl.when(c == 1)
            def _():
                lax.fori_loop(0, gsz // NBUF, make_body(x1_hbm, pg), 0)

            for cp in nxt:
                cp.wait()
            gbase += gsz

        tb = ebase + NFC * CH
        tcol = colb.at[0, pl.ds(0, TAIL)]
        trow = rowb.at[0, pl.ds(0, TAIL)]
        pltpu.sync_copy(e_hbm.at[1, pl.ds(tb, TAIL)], tcol)
        pltpu.sync_copy(e_hbm.at[0, pl.ds(tb, TAIL)], trow)
        tdst = rbuf.at[0, pl.ds(0, TAIL)]

        @pl.when(c == 0)
        def _():
            pltpu.async_copy(x0_hbm.at[tcol], tdst, gsem).wait()

        @pl.when(c == 1)
        def _():
            pltpu.async_copy(x1_hbm.at[tcol], tdst, gsem).wait()

        pltpu.sync_copy(tdst, acc.at[trow], add=True)

        plsc.subcore_barrier()
        sl = pl.ds(s * ROWS_PER_TILE, ROWS_PER_TILE)
        pltpu.sync_copy(acc.at[sl], part_hbm.at[c, sl])

    return k(x0, x1, edges)


def _tc_post(partials, weight, gamma, beta):
    def body(p_ref, w_ref, g_ref, b_ref, o_ref):
        e0, e1, e2, e3, e4, e5, e6, e7 = jnp.split(w_ref[...], 8, axis=1)
        e_0 = jnp.concatenate([e0, -e1, -e2, -e3, -e4, -e5, -e6, -e7], axis=0)
        e_1 = jnp.concatenate([e1, e0, -e3, e2, -e5, e4, e7, -e6], axis=0)
        e_2 = jnp.concatenate([e2, e3, e0, -e1, -e6, -e7, e4, e5], axis=0)
        e_3 = jnp.concatenate([e3, -e2, e1, e0, -e7, e6, -e5, e4], axis=0)
        e_4 = jnp.concatenate([e4, e5, e6, e7, e0, -e1, -e2, -e3], axis=0)
        e_5 = jnp.concatenate([e5, -e4, e7, -e6, e1, e0, e3, -e2], axis=0)
        e_6 = jnp.concatenate([e6, -e7, -e4, e5, e2, -e3, e0, e1], axis=0)
        e_7 = jnp.concatenate([e7, e6, -e5, -e4, e3, e2, -e1, e0], axis=0)
        hamilton = jnp.concatenate([e_0, e_1, e_2, e_3, e_4, e_5, e_6, e_7], axis=1)

        agg = jnp.concatenate([p_ref[0], p_ref[1]], axis=1)
        agg = lax.slice(agg, (0, 0), (N_NODES, F))
        out = jnp.dot(agg, hamilton, preferred_element_type=jnp.float32)
        mean = jnp.mean(out, axis=0, keepdims=True)
        var = jnp.mean(jnp.square(out - mean), axis=0, keepdims=True)
        o_ref[...] = jnp.tanh(
            (out - mean) * lax.rsqrt(var + EPS) * g_ref[...] + b_ref[...]
        )

    return pl.pallas_call(
        body,
        out_shape=jax.ShapeDtypeStruct((N_NODES, F), jnp.float32),
    )(partials, weight, gamma.reshape(1, F), beta.reshape(1, F))


def kernel(x, edge_index, weight, gamma, beta):
    x0 = x[:, :FH]
    x1 = x[:, FH:]
    partials = _sc_spmm(x0, x1, edge_index.astype(jnp.int32))
    return _tc_post(partials, weight, gamma, beta)


# speedup vs baseline: 1.8727x; 1.0172x over previous
import functools

import jax
import jax.numpy as jnp
from jax import lax
from jax.experimental import pallas as pl
from jax.experimental.pallas import tpu as pltpu
from jax.experimental.pallas import tpu_sc as plsc

N_NODES = 10000
N_EDGES = 320000
F = 128
FH = F // 2
EPS = 1e-5

NC = 2
NS = 16
EPT = N_EDGES // NS
CH = 80
NCH = EPT // CH
NBUF = 10
GRP = 50
NG = NCH // GRP
N_PAD = 10240
ROWS_PER_TILE = N_PAD // NS


def _sc_spmm(x0, x1, edges):
    mesh = plsc.VectorSubcoreMesh(core_axis_name="c", subcore_axis_name="s")
    GE = GRP * CH

    @functools.partial(
        pl.kernel,
        mesh=mesh,
        out_type=jax.ShapeDtypeStruct((NC, N_PAD, FH), jnp.float32),
        scratch_types=[
            pltpu.VMEM_SHARED((N_PAD, FH), jnp.float32),
            pltpu.VMEM((2, GE), jnp.int32),
            pltpu.VMEM((2, GE), jnp.int32),
            pltpu.VMEM((NBUF, CH, FH), jnp.float32),
            pltpu.SemaphoreType.DMA,
            pltpu.SemaphoreType.DMA,
            pltpu.SemaphoreType.DMA,
        ],
        compiler_params=pltpu.CompilerParams(use_tc_tiling_on_sc=False),
    )
    def k(x0_hbm, x1_hbm, e_hbm, part_hbm, acc, colb, rowb, rbuf,
          gsem, ssem, isem):
        c = lax.axis_index("c")
        s = lax.axis_index("s")
        ebase = s * EPT

        zc = rbuf.at[0]

        def zbody(i, carry):
            for j in range(FH // 16):
                zc[i, pl.ds(j * 16, 16)] = jnp.zeros((16,), jnp.float32)
            return carry

        lax.fori_loop(0, CH, zbody, 0)
        for t in range(ROWS_PER_TILE // CH):
            pltpu.sync_copy(zc, acc.at[pl.ds(s * ROWS_PER_TILE + t * CH, CH)])

        pltpu.sync_copy(e_hbm.at[1, pl.ds(ebase, GE)], colb.at[0])
        pltpu.sync_copy(e_hbm.at[0, pl.ds(ebase, GE)], rowb.at[0])
        plsc.subcore_barrier()

        def make_body(x_hbm, pg):
            def body(o, carry):
                cps = []
                for b in range(NBUF):
                    cps.append(
                        pltpu.async_copy(
                            x_hbm.at[colb.at[pg, pl.ds((o * NBUF + b) * CH, CH)]],
                            rbuf.at[b], gsem,
                        )
                    )
                scs = []
                for b in range(NBUF):
                    cps[b].wait()
                    scs.append(
                        pltpu.async_copy(
                            rbuf.at[b],
                            acc.at[rowb.at[pg, pl.ds((o * NBUF + b) * CH, CH)]],
                            ssem, add=True,
                        )
                    )
                for b in range(NBUF):
                    scs[b].wait()
                return carry

            return body

        for g in range(NG):
            pg = g % 2
            nxt = []
            if g + 1 < NG:
                nb = ebase + (g + 1) * GE
                nxt.append(
                    pltpu.async_copy(e_hbm.at[1, pl.ds(nb, GE)], colb.at[1 - pg], isem)
                )
                nxt.append(
                    pltpu.async_copy(e_hbm.at[0, pl.ds(nb, GE)], rowb.at[1 - pg], isem)
                )

            @pl.when(c == 0)
            def _():
                lax.fori_loop(0, GRP // NBUF, make_body(x0_hbm, pg), 0)

            @pl.when(c == 1)
            def _():
                lax.fori_loop(0, GRP // NBUF, make_body(x1_hbm, pg), 0)

            for cp in nxt:
                cp.wait()

        plsc.subcore_barrier()
        sl = pl.ds(s * ROWS_PER_TILE, ROWS_PER_TILE)
        pltpu.sync_copy(acc.at[sl], part_hbm.at[c, sl])

    return k(x0, x1, edges)


def _tc_post(partials, weight, gamma, beta):
    def body(p_ref, w_ref, g_ref, b_ref, o_ref):
        e0, e1, e2, e3, e4, e5, e6, e7 = jnp.split(w_ref[...], 8, axis=1)
        e_0 = jnp.concatenate([e0, -e1, -e2, -e3, -e4, -e5, -e6, -e7], axis=0)
        e_1 = jnp.concatenate([e1, e0, -e3, e2, -e5, e4, e7, -e6], axis=0)
        e_2 = jnp.concatenate([e2, e3, e0, -e1, -e6, -e7, e4, e5], axis=0)
        e_3 = jnp.concatenate([e3, -e2, e1, e0, -e7, e6, -e5, e4], axis=0)
        e_4 = jnp.concatenate([e4, e5, e6, e7, e0, -e1, -e2, -e3], axis=0)
        e_5 = jnp.concatenate([e5, -e4, e7, -e6, e1, e0, e3, -e2], axis=0)
        e_6 = jnp.concatenate([e6, -e7, -e4, e5, e2, -e3, e0, e1], axis=0)
        e_7 = jnp.concatenate([e7, e6, -e5, -e4, e3, e2, -e1, e0], axis=0)
        hamilton = jnp.concatenate([e_0, e_1, e_2, e_3, e_4, e_5, e_6, e_7], axis=1)

        agg = jnp.concatenate([p_ref[0], p_ref[1]], axis=1)
        agg = lax.slice(agg, (0, 0), (N_NODES, F))
        out = jnp.dot(agg, hamilton, preferred_element_type=jnp.float32)
        mean = jnp.mean(out, axis=0, keepdims=True)
        var = jnp.mean(jnp.square(out - mean), axis=0, keepdims=True)
        o_ref[...] = jnp.tanh(
            (out - mean) * lax.rsqrt(var + EPS) * g_ref[...] + b_ref[...]
        )

    return pl.pallas_call(
        body,
        out_shape=jax.ShapeDtypeStruct((N_NODES, F), jnp.float32),
    )(partials, weight, gamma.reshape(1, F), beta.reshape(1, F))


def kernel(x, edge_index, weight, gamma, beta):
    x0 = x[:, :FH]
    x1 = x[:, FH:]
    partials = _sc_spmm(x0, x1, edge_index.astype(jnp.int32))
    return _tc_post(partials, weight, gamma, beta)
